# baseline (speedup 1.0000x reference)
import functools

import jax
import jax.numpy as jnp
from jax.experimental import pallas as pl
from jax.experimental.pallas import tpu as pltpu

D = 1024
H = 2048
HALF = D // 2


def _flow_body(x_ref, w1_ref, b1_ref, w2_ref, b2_ref, g1_ref, perm_ref,
               y_ref, ld_ref):
    xb = x_ref[...]
    xb16 = xb.astype(jnp.bfloat16)

    g1 = g1_ref[...]
    rows = jax.lax.broadcasted_iota(jnp.int32, (D, HALF), 0)
    G1 = (rows == g1).astype(jnp.bfloat16)

    x1 = jnp.dot(xb16, G1,
                 preferred_element_type=jnp.float32).astype(jnp.bfloat16)
    h = jnp.tanh(jnp.dot(x1, w1_ref[...],
                         preferred_element_type=jnp.float32) + b1_ref[...])
    params = jnp.dot(h.astype(jnp.bfloat16), w2_ref[...],
                     preferred_element_type=jnp.float32) + b2_ref[...]
    s = jnp.tanh(params[:, :HALF])
    t = params[:, HALF:]

    pm = perm_ref[...]
    jrows = jax.lax.broadcasted_iota(jnp.int32, (HALF, D), 0)
    M = (pm == jrows + HALF).astype(jnp.bfloat16)

    S = jnp.dot(s.astype(jnp.bfloat16), M,
                preferred_element_type=jnp.float32)
    T = jnp.dot(t.astype(jnp.bfloat16), M,
                preferred_element_type=jnp.float32)

    y_ref[...] = xb * jnp.exp(S) + T
    ld_ref[...] = jnp.sum(s, axis=1, keepdims=True)


@functools.partial(jax.jit, static_argnames=("interpret",))
def _run(x, W1, b1, W2, b2, g1_2d, perm_2d, interpret=False):
    N = x.shape[0]
    R = 512
    grid = (N // R,)

    y, ld = pl.pallas_call(
        _flow_body,
        grid=grid,
        in_specs=[
            pl.BlockSpec((R, D), lambda i: (i, 0)),
            pl.BlockSpec((HALF, H), lambda i: (0, 0)),
            pl.BlockSpec((1, H), lambda i: (0, 0)),
            pl.BlockSpec((H, D), lambda i: (0, 0)),
            pl.BlockSpec((1, D), lambda i: (0, 0)),
            pl.BlockSpec((1, HALF), lambda i: (0, 0)),
            pl.BlockSpec((1, D), lambda i: (0, 0)),
        ],
        out_specs=[
            pl.BlockSpec((R, D), lambda i: (i, 0)),
            pl.BlockSpec((R, 1), lambda i: (i, 0)),
        ],
        out_shape=[
            jax.ShapeDtypeStruct((N, D), jnp.float32),
            jax.ShapeDtypeStruct((N, 1), jnp.float32),
        ],
        interpret=interpret,
    )(x, W1.astype(jnp.bfloat16), b1.reshape(1, H),
      W2.astype(jnp.bfloat16), b2.reshape(1, D), g1_2d, perm_2d)
    return y, ld[:, 0]


def kernel(x, W1, b1, W2, b2, perm):
    inv_perm = jnp.argsort(perm).astype(jnp.int32)
    g1_2d = inv_perm[:HALF].reshape(1, HALF)
    perm_2d = perm.astype(jnp.int32).reshape(1, D)
    return _run(x, W1, b1, W2, b2, g1_2d, perm_2d)


# speedup vs baseline: 1.0837x; 1.0837x over previous
import functools

import jax
import jax.numpy as jnp
from jax.experimental import pallas as pl
from jax.experimental.pallas import tpu as pltpu

D = 1024
H = 2048
HALF = D // 2


def _flow_body(x_ref, w1_ref, b1_ref, w2_ref, b2_ref, g1_ref, perm_ref,
               y_ref, ld_ref):
    xb = x_ref[...]

    g1 = g1_ref[...]
    rows = jax.lax.broadcasted_iota(jnp.int32, (D, HALF), 0)
    G1 = (rows == g1).astype(jnp.float32)

    x1 = jnp.dot(xb, G1, preferred_element_type=jnp.float32)
    h = jnp.tanh(jnp.dot(x1, w1_ref[...],
                         preferred_element_type=jnp.float32) + b1_ref[...])
    params = jnp.dot(h, w2_ref[...],
                     preferred_element_type=jnp.float32) + b2_ref[...]
    s = jnp.tanh(params[:, :HALF])
    t = params[:, HALF:]

    pm = perm_ref[...]
    jrows = jax.lax.broadcasted_iota(jnp.int32, (HALF, D), 0)
    M = (pm == jrows + HALF).astype(jnp.float32)

    S = jnp.dot(s, M, preferred_element_type=jnp.float32)
    T = jnp.dot(t, M, preferred_element_type=jnp.float32)

    y_ref[...] = xb * jnp.exp(S) + T
    ld_ref[...] = jnp.sum(s, axis=1, keepdims=True)


@functools.partial(jax.jit, static_argnames=("interpret",))
def _run(x, W1, b1, W2, b2, g1_2d, perm_2d, interpret=False):
    N = x.shape[0]
    R = 512
    grid = (N // R,)

    y, ld = pl.pallas_call(
        _flow_body,
        grid=grid,
        in_specs=[
            pl.BlockSpec((R, D), lambda i: (i, 0)),
            pl.BlockSpec((HALF, H), lambda i: (0, 0)),
            pl.BlockSpec((1, H), lambda i: (0, 0)),
            pl.BlockSpec((H, D), lambda i: (0, 0)),
            pl.BlockSpec((1, D), lambda i: (0, 0)),
            pl.BlockSpec((1, HALF), lambda i: (0, 0)),
            pl.BlockSpec((1, D), lambda i: (0, 0)),
        ],
        out_specs=[
            pl.BlockSpec((R, D), lambda i: (i, 0)),
            pl.BlockSpec((R, 1), lambda i: (i, 0)),
        ],
        out_shape=[
            jax.ShapeDtypeStruct((N, D), jnp.float32),
            jax.ShapeDtypeStruct((N, 1), jnp.float32),
        ],
        interpret=interpret,
    )(x, W1, b1.reshape(1, H), W2, b2.reshape(1, D), g1_2d, perm_2d)
    return y, ld[:, 0]


def kernel(x, W1, b1, W2, b2, perm):
    inv_perm = jnp.argsort(perm).astype(jnp.int32)
    g1_2d = inv_perm[:HALF].reshape(1, HALF)
    perm_2d = perm.astype(jnp.int32).reshape(1, D)
    return _run(x, W1, b1, W2, b2, g1_2d, perm_2d)


# speedup vs baseline: 1.1122x; 1.0263x over previous
import functools

import jax
import jax.numpy as jnp
from jax.experimental import pallas as pl
from jax.experimental.pallas import tpu as pltpu

D = 1024
H = 2048
HALF = D // 2


def _flow_body(x_ref, w1_ref, b1_ref, w2_ref, b2_ref, permc_ref, perm_ref,
               y_ref, ld_ref):
    xb = x_ref[...]

    pc = permc_ref[...]
    cols = jax.lax.broadcasted_iota(jnp.int32, (D, HALF), 1)
    G1 = (pc == cols).astype(jnp.float32)

    x1 = jnp.dot(xb, G1, preferred_element_type=jnp.float32)
    h = jnp.tanh(jnp.dot(x1, w1_ref[...],
                         preferred_element_type=jnp.float32) + b1_ref[...])
    params = jnp.dot(h, w2_ref[...],
                     preferred_element_type=jnp.float32) + b2_ref[...]
    s = jnp.tanh(params[:, :HALF])
    t = params[:, HALF:]

    pm = perm_ref[...]
    jrows = jax.lax.broadcasted_iota(jnp.int32, (HALF, D), 0)
    M = (pm == jrows + HALF).astype(jnp.float32)

    S = jnp.dot(s, M, preferred_element_type=jnp.float32)
    T = jnp.dot(t, M, preferred_element_type=jnp.float32)

    y_ref[...] = xb * jnp.exp(S) + T
    ld_ref[...] = jnp.sum(s, axis=1, keepdims=True)


@functools.partial(jax.jit, static_argnames=("interpret",))
def _run(x, W1, b1, W2, b2, perm_col, perm_2d, interpret=False):
    N = x.shape[0]
    R = 512
    grid = (N // R,)

    y, ld = pl.pallas_call(
        _flow_body,
        grid=grid,
        in_specs=[
            pl.BlockSpec((R, D), lambda i: (i, 0)),
            pl.BlockSpec((HALF, H), lambda i: (0, 0)),
            pl.BlockSpec((1, H), lambda i: (0, 0)),
            pl.BlockSpec((H, D), lambda i: (0, 0)),
            pl.BlockSpec((1, D), lambda i: (0, 0)),
            pl.BlockSpec((D, 1), lambda i: (0, 0)),
            pl.BlockSpec((1, D), lambda i: (0, 0)),
        ],
        out_specs=[
            pl.BlockSpec((R, D), lambda i: (i, 0)),
            pl.BlockSpec((R, 1), lambda i: (i, 0)),
        ],
        out_shape=[
            jax.ShapeDtypeStruct((N, D), jnp.float32),
            jax.ShapeDtypeStruct((N, 1), jnp.float32),
        ],
        interpret=interpret,
    )(x, W1, b1.reshape(1, H), W2, b2.reshape(1, D), perm_col, perm_2d)
    return y, ld[:, 0]


def kernel(x, W1, b1, W2, b2, perm):
    perm = perm.astype(jnp.int32)
    return _run(x, W1, b1, W2, b2, perm.reshape(D, 1), perm.reshape(1, D))


# speedup vs baseline: 1.1238x; 1.0104x over previous
import functools

import jax
import jax.numpy as jnp
from jax.experimental import pallas as pl
from jax.experimental.pallas import tpu as pltpu

D = 1024
H = 2048
HALF = D // 2


def _flow_body(x_ref, w1_ref, b1_ref, w2_ref, b2_ref, permc_ref, perm_ref,
               y_ref, ld_ref):
    xb = x_ref[...]

    pc = permc_ref[...]
    cols = jax.lax.broadcasted_iota(jnp.int32, (D, HALF), 1)
    G1 = (pc == cols).astype(jnp.float32)

    x1 = jnp.dot(xb, G1, preferred_element_type=jnp.float32)
    h = jnp.tanh(jnp.dot(x1, w1_ref[...],
                         preferred_element_type=jnp.float32) + b1_ref[...])
    params = jnp.dot(h, w2_ref[...],
                     preferred_element_type=jnp.float32) + b2_ref[...]
    s = jnp.tanh(params[:, :HALF])
    t = params[:, HALF:]

    pm = perm_ref[...]
    jrows = jax.lax.broadcasted_iota(jnp.int32, (HALF, D), 0)
    M = (pm == jrows + HALF).astype(jnp.float32)

    S = jnp.dot(s, M, preferred_element_type=jnp.float32)
    T = jnp.dot(t, M, preferred_element_type=jnp.float32)

    y_ref[...] = xb * jnp.exp(S) + T
    ld_ref[...] = jnp.sum(s, axis=1, keepdims=True)


@functools.partial(jax.jit, static_argnames=("interpret",))
def _run(x, W1, b1, W2, b2, perm_col, perm_2d, interpret=False):
    N = x.shape[0]
    R = 1024
    grid = (N // R,)

    y, ld = pl.pallas_call(
        _flow_body,
        grid=grid,
        in_specs=[
            pl.BlockSpec((R, D), lambda i: (i, 0)),
            pl.BlockSpec((HALF, H), lambda i: (0, 0)),
            pl.BlockSpec((1, H), lambda i: (0, 0)),
            pl.BlockSpec((H, D), lambda i: (0, 0)),
            pl.BlockSpec((1, D), lambda i: (0, 0)),
            pl.BlockSpec((D, 1), lambda i: (0, 0)),
            pl.BlockSpec((1, D), lambda i: (0, 0)),
        ],
        out_specs=[
            pl.BlockSpec((R, D), lambda i: (i, 0)),
            pl.BlockSpec((R, 1), lambda i: (i, 0)),
        ],
        out_shape=[
            jax.ShapeDtypeStruct((N, D), jnp.float32),
            jax.ShapeDtypeStruct((N, 1), jnp.float32),
        ],
        interpret=interpret,
    )(x, W1, b1.reshape(1, H), W2, b2.reshape(1, D), perm_col, perm_2d)
    return y, ld[:, 0]


def kernel(x, W1, b1, W2, b2, perm):
    perm = perm.astype(jnp.int32)
    return _run(x, W1, b1, W2, b2, perm.reshape(D, 1), perm.reshape(1, D))
